# baseline (speedup 1.0000x reference)
import functools

import jax
import jax.numpy as jnp
from jax.experimental import pallas as pl
from jax.experimental.pallas import tpu as pltpu

_B, _N, _D = 4, 4096, 3
_STRIP = 64
_NSTRIPS = _N // _STRIP
_YG = _N // 128


def _rnd(a):
    return a.astype(jnp.bfloat16).astype(jnp.float32)


def _chamfer_body(x_ref, yr_ref, out_ref, yb_ref, ysq_ref):
    b = pl.program_id(0)

    y0 = yr_ref[0, 0]
    y1 = yr_ref[0, 1]
    y2 = yr_ref[0, 2]
    yb_ref[0] = _rnd(y0)
    yb_ref[1] = _rnd(y1)
    yb_ref[2] = _rnd(y2)
    ysq_ref[...] = y0 * y0 + y1 * y1 + y2 * y2

    def strip_loop(s, total):
        xs = x_ref[0, pl.ds(s * _STRIP, _STRIP), :]
        xb0 = jnp.broadcast_to(_rnd(xs[:, 0:1]), (_STRIP, 128))
        xb1 = jnp.broadcast_to(_rnd(xs[:, 1:2]), (_STRIP, 128))
        xb2 = jnp.broadcast_to(_rnd(xs[:, 2:3]), (_STRIP, 128))

        def ygroup_loop(j, acc):
            yj0 = yb_ref[0, pl.ds(j, 1), :]
            yj1 = yb_ref[1, pl.ds(j, 1), :]
            yj2 = yb_ref[2, pl.ds(j, 1), :]
            yjs = ysq_ref[pl.ds(j, 1), :]
            xy = xb0 * yj0 + xb1 * yj1 + xb2 * yj2
            t = xy * -2.0 + yjs
            return jnp.minimum(acc, t)

        acc0 = jnp.full((_STRIP, 128), jnp.inf, dtype=jnp.float32)
        acc = jax.lax.fori_loop(0, _YG, ygroup_loop, acc0)
        m = jnp.min(acc, axis=1, keepdims=True)
        xsq = xs[:, 0:1] * xs[:, 0:1] + xs[:, 1:2] * xs[:, 1:2] \
            + xs[:, 2:3] * xs[:, 2:3]
        d = jnp.maximum(m + xsq, 0.0)
        return total + jnp.sum(d)

    bsum = jax.lax.fori_loop(0, _NSTRIPS, strip_loop, jnp.float32(0.0))

    @pl.when(b == 0)
    def _init():
        out_ref[0, 0] = 0.0

    out_ref[0, 0] += bsum


@functools.partial(jax.jit, static_argnames=())
def kernel(v, v_pred):
    yr = jnp.transpose(v, (0, 2, 1)).reshape(_B, _D, _YG, 128)
    out = pl.pallas_call(
        _chamfer_body,
        grid=(_B,),
        in_specs=[
            pl.BlockSpec((1, _N, _D), lambda b: (b, 0, 0)),
            pl.BlockSpec((1, _D, _YG, 128), lambda b: (b, 0, 0, 0)),
        ],
        out_specs=pl.BlockSpec(
            (1, 1), lambda b: (0, 0), memory_space=pltpu.SMEM
        ),
        out_shape=jax.ShapeDtypeStruct((1, 1), jnp.float32),
        scratch_shapes=[
            pltpu.VMEM((_D, _YG, 128), jnp.float32),
            pltpu.VMEM((_YG, 128), jnp.float32),
        ],
    )(v_pred, yr)
    return out[0, 0] * (1.0 / (_B * _N))


# speedup vs baseline: 1.4275x; 1.4275x over previous
import functools

import jax
import jax.numpy as jnp
from jax.experimental import pallas as pl
from jax.experimental.pallas import tpu as pltpu

_B, _N, _D = 4, 4096, 3
_STRIP = 64
_NSTRIPS = _N // _STRIP
_YG = _N // 128


def _rnd(a):
    return a.astype(jnp.bfloat16).astype(jnp.float32)


def _chamfer_body(x_ref, yr_ref, out_ref, yb_ref, ysq_ref):
    b = pl.program_id(0)

    y0 = yr_ref[0, 0]
    y1 = yr_ref[0, 1]
    y2 = yr_ref[0, 2]
    yb_ref[0] = _rnd(y0)
    yb_ref[1] = _rnd(y1)
    yb_ref[2] = _rnd(y2)
    ysq_ref[...] = y0 * y0 + y1 * y1 + y2 * y2

    def strip_loop(s, total):
        xs = x_ref[0, pl.ds(s * _STRIP, _STRIP), :]
        nx0 = jnp.broadcast_to(_rnd(xs[:, 0:1]) * -2.0, (_STRIP, 128))
        nx1 = jnp.broadcast_to(_rnd(xs[:, 1:2]) * -2.0, (_STRIP, 128))
        nx2 = jnp.broadcast_to(_rnd(xs[:, 2:3]) * -2.0, (_STRIP, 128))

        acc = jnp.full((_STRIP, 128), jnp.inf, dtype=jnp.float32)
        for j in range(_YG):
            yj0 = yb_ref[0, j : j + 1, :]
            yj1 = yb_ref[1, j : j + 1, :]
            yj2 = yb_ref[2, j : j + 1, :]
            t = ysq_ref[j : j + 1, :] + nx0 * yj0
            t = t + nx1 * yj1
            t = t + nx2 * yj2
            acc = jnp.minimum(acc, t)
        m = jnp.min(acc, axis=1, keepdims=True)
        xsq = xs[:, 0:1] * xs[:, 0:1] + xs[:, 1:2] * xs[:, 1:2] \
            + xs[:, 2:3] * xs[:, 2:3]
        d = jnp.maximum(m + xsq, 0.0)
        return total + jnp.sum(d)

    bsum = jax.lax.fori_loop(0, _NSTRIPS, strip_loop, jnp.float32(0.0))

    @pl.when(b == 0)
    def _init():
        out_ref[0, 0] = 0.0

    out_ref[0, 0] += bsum


@functools.partial(jax.jit, static_argnames=())
def kernel(v, v_pred):
    yr = jnp.transpose(v, (0, 2, 1)).reshape(_B, _D, _YG, 128)
    out = pl.pallas_call(
        _chamfer_body,
        grid=(_B,),
        in_specs=[
            pl.BlockSpec((1, _N, _D), lambda b: (b, 0, 0)),
            pl.BlockSpec((1, _D, _YG, 128), lambda b: (b, 0, 0, 0)),
        ],
        out_specs=pl.BlockSpec(
            (1, 1), lambda b: (0, 0), memory_space=pltpu.SMEM
        ),
        out_shape=jax.ShapeDtypeStruct((1, 1), jnp.float32),
        scratch_shapes=[
            pltpu.VMEM((_D, _YG, 128), jnp.float32),
            pltpu.VMEM((_YG, 128), jnp.float32),
        ],
    )(v_pred, yr)
    return out[0, 0] * (1.0 / (_B * _N))


# speedup vs baseline: 3.3353x; 2.3365x over previous
import jax
import jax.numpy as jnp
from jax.experimental import pallas as pl
from jax.experimental.pallas import tpu as pltpu

_B, _N, _D = 4, 4096, 3
_STRIP = 256
_NSTRIPS = _N // _STRIP
_YG = _N // 128


def _chamfer_body(x_ref, yr_ref, out_ref, ysq_ref, yb_ref, macc_ref):
    b = pl.program_id(0)

    y0 = yr_ref[0, 0]
    y1 = yr_ref[0, 1]
    y2 = yr_ref[0, 2]
    ysq_ref[...] = y0 * y0 + y1 * y1 + y2 * y2
    yb_ref[0] = y0.astype(jnp.bfloat16)
    yb_ref[1] = y1.astype(jnp.bfloat16)
    yb_ref[2] = y2.astype(jnp.bfloat16)

    def strip_loop(s, carry):
        xs = x_ref[0, pl.ds(s * _STRIP, _STRIP), :]
        xw = (xs * -2.0).astype(jnp.bfloat16)

        acc = jnp.full((_STRIP, 128), jnp.inf, dtype=jnp.float32)
        for m in range(_YG):
            w = yb_ref[:, m, :]
            g = jax.lax.dot_general(
                xw, w, (((1,), (0,)), ((), ())),
                preferred_element_type=jnp.float32,
            )
            t = g + ysq_ref[m : m + 1, :]
            acc = jnp.minimum(acc, t)

        macc_ref[pl.ds(s * _STRIP, _STRIP), :] = acc
        return carry

    jax.lax.fori_loop(0, _NSTRIPS, strip_loop, jnp.float32(0.0))

    m0 = jnp.min(macc_ref[...], axis=1, keepdims=True)
    xsf = x_ref[0]
    xsq = xsf[:, 0:1] * xsf[:, 0:1] + xsf[:, 1:2] * xsf[:, 1:2] \
        + xsf[:, 2:3] * xsf[:, 2:3]
    bsum = jnp.sum(jnp.maximum(m0 + xsq, 0.0))

    @pl.when(b == 0)
    def _init():
        out_ref[0, 0] = 0.0

    out_ref[0, 0] += bsum


def kernel(v, v_pred):
    yr = jnp.transpose(v, (0, 2, 1)).reshape(_B, _D, _YG, 128)
    out = pl.pallas_call(
        _chamfer_body,
        grid=(_B,),
        in_specs=[
            pl.BlockSpec((1, _N, _D), lambda b: (b, 0, 0)),
            pl.BlockSpec((1, _D, _YG, 128), lambda b: (b, 0, 0, 0)),
        ],
        out_specs=pl.BlockSpec(
            (1, 1), lambda b: (0, 0), memory_space=pltpu.SMEM
        ),
        out_shape=jax.ShapeDtypeStruct((1, 1), jnp.float32),
        scratch_shapes=[
            pltpu.VMEM((_YG, 128), jnp.float32),
            pltpu.VMEM((_D, _YG, 128), jnp.bfloat16),
            pltpu.VMEM((_N, 128), jnp.float32),
        ],
    )(v_pred, yr)
    return out[0, 0] * (1.0 / (_B * _N))
